# Initial kernel scaffold; baseline (speedup 1.0000x reference)
import jax
import jax.numpy as jnp
from jax.experimental import pallas as pl


def kernel(lift_fresh_copy_1, index_put_1, view):
    raise NotImplementedError("write your pallas kernel here")


# speedup vs baseline: 21.7794x; 21.7794x over previous
import functools

import jax
import jax.numpy as jnp
from jax import lax
from jax.experimental import pallas as pl
from jax.experimental.pallas import tpu as pltpu
from jax.experimental.pallas import tpu_sc as plsc

N_IDX = 16384
N_MEM = 1000000
D = 64
NC = 2
NS = 16
L = 16
NW = NC * NS
DOM = 31264
CHUNKS = N_IDX // L
GCH = 128
ROWS = N_IDX // GCH + 1


def _body(idx_hbm, val_hbm, out_hbm, idx_v, tloc, nb, pos2, wp2, rows_v, sem):
    c = lax.axis_index("c")
    s = lax.axis_index("s")
    w = s * NC + c
    lo = w * DOM
    lane = lax.iota(jnp.int32, L)

    pltpu.sync_copy(idx_hbm, idx_v)

    nb[pl.ds(L, L)] = jnp.full((L,), -1, jnp.int32)

    def scat(i, carry):
        off = pl.multiple_of(i * L, L)
        iv = idx_v[pl.ds(off, L)]
        nb[pl.ds(0, L)] = iv
        m = (iv >= lo) & (iv < lo + DOM)
        for sh in range(1, L):
            m = m & (iv != nb[pl.ds(sh, L)])
        tgt = jnp.where(m, iv - lo, 0)
        plsc.store_scatter(tloc, [tgt], off + lane, mask=m)
        return carry

    lax.fori_loop(0, CHUNKS, scat, 0)

    def comp(i, n_vec):
        off = pl.multiple_of(i * L, L)
        iv = idx_v[pl.ds(off, L)]
        m = (iv >= lo) & (iv < lo + DOM)
        loc = jnp.where(m, iv - lo, 0)
        wv = plsc.load_gather(tloc, [loc], mask=m)
        tgt = n_vec + plsc.cumsum(m.astype(jnp.int32)) - 1
        tgt = jnp.where(m, tgt, 0)
        plsc.store_scatter(pos2, [tgt >> 7, tgt & (GCH - 1)], off + lane,
                           mask=m)
        plsc.store_scatter(wp2, [tgt >> 7, tgt & (GCH - 1)], wv, mask=m)
        return n_vec + plsc.all_reduce_population_count(m)

    n_vec = lax.fori_loop(0, CHUNKS, comp, jnp.zeros((L,), jnp.int32))
    n = jnp.max(n_vec)

    z = jnp.zeros((L,), jnp.int32)
    p0 = plsc.load_gather(pos2, [z, z])
    w0 = plsc.load_gather(wp2, [z, z])
    for t in range(GCH // L):
        tgt = n + t * L + lane
        plsc.store_scatter(pos2, [tgt >> 7, tgt & (GCH - 1)], p0)
        plsc.store_scatter(wp2, [tgt >> 7, tgt & (GCH - 1)], w0)

    nch = (n + GCH - 1) >> 7

    def emit(j, carry):
        pltpu.async_copy(val_hbm.at[wp2.at[j]], rows_v, sem).wait()
        pltpu.async_copy(rows_v, out_hbm.at[pos2.at[j]], sem).wait()
        return carry

    lax.fori_loop(0, nch, emit, 0)


@jax.jit
def _run(idx, val):
    mesh = plsc.VectorSubcoreMesh(core_axis_name="c", subcore_axis_name="s")
    f = functools.partial(
        pl.kernel,
        mesh=mesh,
        compiler_params=pltpu.CompilerParams(
            needs_layout_passes=False, use_tc_tiling_on_sc=False),
        out_type=jax.ShapeDtypeStruct((N_IDX, D), jnp.float32),
        scratch_types=[
            pltpu.VMEM((N_IDX,), jnp.int32),
            pltpu.VMEM((DOM,), jnp.int32),
            pltpu.VMEM((2 * L,), jnp.int32),
            pltpu.VMEM((ROWS, GCH), jnp.int32),
            pltpu.VMEM((ROWS, GCH), jnp.int32),
            pltpu.VMEM((GCH, D), jnp.float32),
            pltpu.SemaphoreType.DMA,
        ],
    )(_body)
    return f(idx, val)


def kernel(lift_fresh_copy_1, index_put_1, view):
    del index_put_1
    idx = lift_fresh_copy_1.astype(jnp.int32)
    return _run(idx, view)
